# baseline (speedup 1.0000x reference)
import functools
from math import pi as PI

import numpy as np
import jax
import jax.numpy as jnp
from jax import lax
from jax.experimental import pallas as pl
from jax.experimental.pallas import tpu as pltpu
from jax.experimental.pallas import tpu_sc as plsc

N = 10000
E = 320000
H = 128
F = 128
G = 50
L = 6
CUTOFF = 10.0
NG = 64

_NC = 2
_NS = 16
_NW = _NC * _NS
_EPW = E // _NW
_CH = 80
_NCHUNK = _EPW // _CH
_NP = 10240
_LOG2 = 0.6931471805599453
_STEP = CUTOFF / (G - 1)
_COEFF = -0.5 / _STEP ** 2

_mesh = plsc.VectorSubcoreMesh(
    core_axis_name="c", subcore_axis_name="s", num_cores=_NC, num_subcores=_NS
)


@functools.partial(
    pl.kernel,
    out_type=jax.ShapeDtypeStruct((E,), jnp.float32),
    mesh=_mesh,
    compiler_params=pltpu.CompilerParams(needs_layout_passes=False),
    scratch_types=[
        pltpu.VMEM((N,), jnp.float32),
        pltpu.VMEM((N,), jnp.float32),
        pltpu.VMEM((N,), jnp.float32),
        pltpu.VMEM((_EPW,), jnp.int32),
        pltpu.VMEM((_EPW,), jnp.int32),
        pltpu.VMEM((_EPW,), jnp.float32),
    ],
)
def _d2_kernel(posx_hbm, posy_hbm, posz_hbm, row_hbm, col_hbm, out_hbm,
               px, py, pz, rv, cv, dv):
    cid = lax.axis_index("c")
    sid = lax.axis_index("s")
    wid = sid * _NC + cid
    base = wid * _EPW
    pltpu.sync_copy(posx_hbm, px)
    pltpu.sync_copy(posy_hbm, py)
    pltpu.sync_copy(posz_hbm, pz)
    pltpu.sync_copy(row_hbm.at[pl.ds(base, _EPW)], rv)
    pltpu.sync_copy(col_hbm.at[pl.ds(base, _EPW)], cv)

    @plsc.parallel_loop(0, _EPW // 16, 1, unroll=2)
    def _(k):
        o = k * 16
        ir = rv[pl.ds(o, 16)]
        ic = cv[pl.ds(o, 16)]
        dx = plsc.load_gather(px, [ir]) - plsc.load_gather(px, [ic])
        dy = plsc.load_gather(py, [ir]) - plsc.load_gather(py, [ic])
        dz = plsc.load_gather(pz, [ir]) - plsc.load_gather(pz, [ic])
        dv[pl.ds(o, 16)] = dx * dx + dy * dy + dz * dz

    pltpu.sync_copy(dv, out_hbm.at[pl.ds(base, _EPW)])


@functools.partial(
    pl.kernel,
    out_type=jax.ShapeDtypeStruct((_NC, _NP, H), jnp.float32),
    mesh=_mesh,
    compiler_params=pltpu.CompilerParams(needs_layout_passes=False),
    scratch_types=[
        pltpu.VMEM_SHARED((_NP, H), jnp.float32),
        pltpu.VMEM((_CH,), jnp.int32),
        pltpu.VMEM((_CH,), jnp.int32),
        pltpu.VMEM((_CH,), jnp.int32),
        pltpu.VMEM((_CH,), jnp.int32),
        pltpu.VMEM((_CH, H // 2), jnp.int32),
        pltpu.VMEM((_CH, H // 2), jnp.int32),
        pltpu.VMEM((_CH, H), jnp.float32),
        pltpu.VMEM((_CH, H), jnp.float32),
        pltpu.SemaphoreType.DMA,
        pltpu.SemaphoreType.DMA,
        pltpu.SemaphoreType.DMA,
        pltpu.SemaphoreType.DMA,
        pltpu.SemaphoreType.DMA,
        pltpu.SemaphoreType.DMA,
        pltpu.SemaphoreType.DMA,
        pltpu.SemaphoreType.DMA,
    ],
)
def _cfconv(x1_hbm, wf_hbm, row_hbm, col_hbm, zeros_hbm, out_hbm,
            acc, rva, rvb, cva, cvb, wva, wvb, mva, mvb,
            sra, srb, sca, scb, swa, swb, sga, sgb):
    cid = lax.axis_index("c")
    sid = lax.axis_index("s")
    wid = sid * _NC + cid
    base = wid * _EPW
    rps = _NP // _NS
    pltpu.sync_copy(zeros_hbm.at[pl.ds(sid * rps, rps)],
                    acc.at[pl.ds(sid * rps, rps)])
    plsc.subcore_barrier()

    bufa = (rva, cva, wva, mva, sra, sca, swa, sga)
    bufb = (rvb, cvb, wvb, mvb, srb, scb, swb, sgb)

    def issue_data(t, buf):
        rv, cv, wv, mv, sr, sc, sw, sg = buf
        cb = base + t * _CH
        pltpu.async_copy(x1_hbm.at[rv], mv, sg)
        pltpu.async_copy(col_hbm.at[pl.ds(cb, _CH)], cv, sc)
        pltpu.async_copy(wf_hbm.at[pl.ds(cb, _CH)], wv, sw)

    def issue_row(t, buf):
        rv, cv, wv, mv, sr, sc, sw, sg = buf
        pltpu.async_copy(row_hbm.at[pl.ds(base + t * _CH, _CH)], rv, sr)

    def wait_row(buf):
        rv, cv, wv, mv, sr, sc, sw, sg = buf
        pltpu.make_async_copy(row_hbm.at[pl.ds(base, _CH)], rv, sr).wait()

    def phase(t, cur, nxt):
        rv, cv, wv, mv, sr, sc, sw, sg = cur

        @pl.when(t < _NCHUNK)
        def _():
            pltpu.make_async_copy(x1_hbm.at[rv], mv, sg).wait()
            pltpu.make_async_copy(col_hbm.at[pl.ds(base, _CH)], cv, sc).wait()
            pltpu.make_async_copy(wf_hbm.at[pl.ds(0, _CH)], wv, sw).wait()

            @pl.when(t + 1 < _NCHUNK)
            def _():
                wait_row(nxt)
                issue_data(t + 1, nxt)

            @pl.when(t + 2 < _NCHUNK)
            def _():
                issue_row(t + 2, cur)

            @plsc.parallel_loop(0, _CH, 1, unroll=2)
            def _(i):
                for j in range(H // 32):
                    wi = wv[i, pl.ds(j * 16, 16)]
                    wa = plsc.bitcast(wi << 16, jnp.float32)
                    wb = plsc.bitcast(wi & jnp.int32(-65536), jnp.float32)
                    s0 = pl.ds(j * 16, 16)
                    s1 = pl.ds(H // 2 + j * 16, 16)
                    mv[i, s0] = mv[i, s0] * wa
                    mv[i, s1] = mv[i, s1] * wb

            pltpu.sync_copy(mv, acc.at[cv], add=True)

    pltpu.sync_copy(row_hbm.at[pl.ds(base, _CH)], rva)
    issue_data(0, bufa)
    issue_row(1, bufb)

    def body(g, carry):
        phase(2 * g, bufa, bufb)
        phase(2 * g + 1, bufb, bufa)
        return carry

    lax.fori_loop(0, (_NCHUNK + 1) // 2, body, 0)
    plsc.subcore_barrier()
    pltpu.sync_copy(acc.at[pl.ds(sid * rps, rps)],
                    out_hbm.at[cid, pl.ds(sid * rps, rps)])


_BLKE = 640
_BLKN = 400
_ELR = E // 128
_BLKR = _ELR


def _prep_body(d2_ref, out_ref):
    ew = jnp.sqrt(d2_ref[:])
    out_ref[:] = 0.5 * (jnp.cos(ew * (PI / CUTOFF)) + 1.0)


def _prep(d2lanes):
    shp = (E // _BLKE, _BLKE // 128, 128)
    return pl.pallas_call(
        _prep_body,
        grid=(1,),
        in_specs=[pl.BlockSpec(shp, lambda i: (0, 0, 0))],
        out_specs=pl.BlockSpec(shp, lambda i: (0, 0, 0)),
        out_shape=jax.ShapeDtypeStruct(shp, jnp.float32),
    )(d2lanes)


def _tocol(blk):
    i0 = lax.broadcasted_iota(jnp.int32, (128, 128), 0)
    i1 = lax.broadcasted_iota(jnp.int32, (128, 128), 1)
    ident = (i0 == i1).astype(jnp.float32)
    parts = [lax.dot_general(ident, blk[s:s + 1, :], (((1,), (1,)), ((), ())))
             for s in range(_BLKE // 128)]
    return jnp.concatenate(parts, axis=0)


def _filters_body(d2_ref, c_ref, w1_ref, b1_ref, w2a_ref, b2a_ref,
                  w2b_ref, b2b_ref, out_ref):
    ew = _tocol(jnp.sqrt(d2_ref[0]))
    c = _tocol(c_ref[0])
    offs = lax.broadcasted_iota(jnp.int32, (1, G), 1).astype(jnp.float32) * _STEP
    ga = jnp.exp(_COEFF * (ew - offs) ** 2).astype(jnp.bfloat16)
    dn = (((1,), (0,)), ((), ()))
    t0 = lax.dot_general(ga, w1_ref[:], dn, preferred_element_type=jnp.float32)
    t = (jax.nn.softplus(t0 + b1_ref[:]) - _LOG2).astype(jnp.bfloat16)
    wa = lax.dot_general(t, w2a_ref[:], dn, preferred_element_type=jnp.float32)
    wb = lax.dot_general(t, w2b_ref[:], dn, preferred_element_type=jnp.float32)
    wfa = ((wa + b2a_ref[:]) * c).astype(jnp.bfloat16)
    wfb = ((wb + b2b_ref[:]) * c).astype(jnp.bfloat16)
    lo = lax.bitcast_convert_type(wfa, jnp.uint16).astype(jnp.int32)
    hi = lax.bitcast_convert_type(wfb, jnp.uint16).astype(jnp.int32)
    out_ref[:] = lo | (hi << 16)


def _filters_one(d2lanes, cenvlanes, w1, b1, w2a, b2a, w2b, b2b):
    return pl.pallas_call(
        _filters_body,
        grid=(E // _BLKE,),
        in_specs=[
            pl.BlockSpec((1, _BLKE // 128, 128), lambda e: (e, 0, 0)),
            pl.BlockSpec((1, _BLKE // 128, 128), lambda e: (e, 0, 0)),
            pl.BlockSpec((G, F), lambda e: (0, 0)),
            pl.BlockSpec((1, F), lambda e: (0, 0)),
            pl.BlockSpec((F, F // 2), lambda e: (0, 0)),
            pl.BlockSpec((1, F // 2), lambda e: (0, 0)),
            pl.BlockSpec((F, F // 2), lambda e: (0, 0)),
            pl.BlockSpec((1, F // 2), lambda e: (0, 0)),
        ],
        out_specs=pl.BlockSpec((_BLKE, F // 2), lambda e: (e, 0)),
        out_shape=jax.ShapeDtypeStruct((E, F // 2), jnp.int32),
    )(d2lanes, cenvlanes, w1, b1, w2a, b2a, w2b, b2b)


def _embed_body(a_ref, emb_ref, w_ref, h_ref, x1_ref):
    a = a_ref[:]
    h0 = jnp.where(a == 0, emb_ref[0][None, :], emb_ref[1][None, :])
    h_ref[:] = h0
    x1_ref[:] = h0 @ w_ref[:]


def _embed(a32, emb, w0):
    return pl.pallas_call(
        _embed_body,
        grid=(N // _BLKN,),
        in_specs=[
            pl.BlockSpec((_BLKN, 1), lambda i: (i, 0)),
            pl.BlockSpec((2, H), lambda i: (0, 0)),
            pl.BlockSpec((H, F), lambda i: (0, 0)),
        ],
        out_specs=[
            pl.BlockSpec((_BLKN, H), lambda i: (i, 0)),
            pl.BlockSpec((_BLKN, F), lambda i: (i, 0)),
        ],
        out_shape=[
            jax.ShapeDtypeStruct((N, H), jnp.float32),
            jax.ShapeDtypeStruct((N, F), jnp.float32),
        ],
    )(a32, emb, w0)


def _update_body(h_ref, agg_ref, w2_ref, b2_ref, wl_ref, bl_ref, wn_ref,
                 h_out, x1_out):
    agg = agg_ref[0] + agg_ref[1]
    x2 = jax.nn.softplus(agg @ w2_ref[:] + b2_ref[:]) - _LOG2
    x2 = x2 @ wl_ref[:] + bl_ref[:]
    hn = h_ref[:] + x2
    h_out[:] = hn
    x1_out[:] = hn @ wn_ref[:]


def _update(h, agg2, w2, b2, wl, bl, wn):
    return pl.pallas_call(
        _update_body,
        grid=(N // _BLKN,),
        in_specs=[
            pl.BlockSpec((_BLKN, H), lambda i: (i, 0)),
            pl.BlockSpec((_NC, _BLKN, H), lambda i: (0, i, 0)),
            pl.BlockSpec((F, H), lambda i: (0, 0)),
            pl.BlockSpec((1, H), lambda i: (0, 0)),
            pl.BlockSpec((H, H), lambda i: (0, 0)),
            pl.BlockSpec((1, H), lambda i: (0, 0)),
            pl.BlockSpec((H, F), lambda i: (0, 0)),
        ],
        out_specs=[
            pl.BlockSpec((_BLKN, H), lambda i: (i, 0)),
            pl.BlockSpec((_BLKN, F), lambda i: (i, 0)),
        ],
        out_shape=[
            jax.ShapeDtypeStruct((N, H), jnp.float32),
            jax.ShapeDtypeStruct((N, F), jnp.float32),
        ],
    )(h, agg2, w2, b2, wl, bl, wn)


def _head_body(h_ref, b_ref, w1_ref, b1_ref, ew_ref, eb_ref, qw_ref, qb_ref,
               e_ref, q_ref, et_ref, qt_ref):
    i = pl.program_id(0)
    hh = jax.nn.softplus(h_ref[:] @ w1_ref[:] + b1_ref[:]) - _LOG2
    e = hh @ ew_ref[:] + eb_ref[:]
    q = hh @ qw_ref[:] + qb_ref[:]
    e_ref[:] = e
    q_ref[:] = q
    b = b_ref[:]
    onehot = (b == lax.broadcasted_iota(jnp.int32, (1, NG), 1)).astype(jnp.float32)
    et_p = lax.dot_general(onehot, e, (((0,), (0,)), ((), ())))
    qt_p = lax.dot_general(onehot, q, (((0,), (0,)), ((), ())))

    @pl.when(i == 0)
    def _():
        et_ref[:] = jnp.zeros_like(et_ref)
        qt_ref[:] = jnp.zeros_like(qt_ref)

    et_ref[:] += et_p
    qt_ref[:] += qt_p


def _head(h, b32, w1, b1, ew, eb, qw, qb):
    return pl.pallas_call(
        _head_body,
        grid=(N // _BLKN,),
        in_specs=[
            pl.BlockSpec((_BLKN, H), lambda i: (i, 0)),
            pl.BlockSpec((_BLKN, 1), lambda i: (i, 0)),
            pl.BlockSpec((H, H // 2), lambda i: (0, 0)),
            pl.BlockSpec((1, H // 2), lambda i: (0, 0)),
            pl.BlockSpec((H // 2, 1), lambda i: (0, 0)),
            pl.BlockSpec((1, 1), lambda i: (0, 0)),
            pl.BlockSpec((H // 2, 1), lambda i: (0, 0)),
            pl.BlockSpec((1, 1), lambda i: (0, 0)),
        ],
        out_specs=[
            pl.BlockSpec((_BLKN, 1), lambda i: (i, 0)),
            pl.BlockSpec((_BLKN, 1), lambda i: (i, 0)),
            pl.BlockSpec((NG, 1), lambda i: (0, 0)),
            pl.BlockSpec((NG, 1), lambda i: (0, 0)),
        ],
        out_shape=[
            jax.ShapeDtypeStruct((N, 1), jnp.float32),
            jax.ShapeDtypeStruct((N, 1), jnp.float32),
            jax.ShapeDtypeStruct((NG, 1), jnp.float32),
            jax.ShapeDtypeStruct((NG, 1), jnp.float32),
        ],
    )(h, b32, w1, b1, ew, eb, qw, qb)


def kernel(pos, params, atoms, batch_idx, edge_index):
    row = edge_index[0].astype(jnp.int32)
    col = edge_index[1].astype(jnp.int32)
    a32 = atoms.astype(jnp.int32).reshape(N, 1)
    b32 = batch_idx.astype(jnp.int32).reshape(N, 1)
    posx = pos[:, 0]
    posy = pos[:, 1]
    posz = pos[:, 2]

    d2 = _d2_kernel(posx, posy, posz, row, col)
    d2lanes = d2.reshape(E // _BLKE, _BLKE // 128, 128)
    cenvlanes = _prep(d2lanes)

    lp = params["layers"]
    h, x1 = _embed(a32, params["emb"], lp[0]["conv_lin1_w"])
    zeros_nh = jnp.zeros((_NP, H), jnp.float32)
    for l in range(L):
        w2 = lp[l]["mlp_w2"].astype(jnp.bfloat16)
        b2 = lp[l]["mlp_b2"]
        wf = _filters_one(d2lanes, cenvlanes,
                          lp[l]["mlp_w1"].astype(jnp.bfloat16),
                          lp[l]["mlp_b1"].reshape(1, F),
                          w2[:, :F // 2], b2[:F // 2].reshape(1, F // 2),
                          w2[:, F // 2:], b2[F // 2:].reshape(1, F // 2))
        agg2 = _cfconv(x1, wf, row, col, zeros_nh)
        wn = (lp[l + 1]["conv_lin1_w"] if l < L - 1
              else jnp.zeros((H, F), jnp.float32))
        h, x1 = _update(h, agg2,
                        lp[l]["conv_lin2_w"], lp[l]["conv_lin2_b"].reshape(1, H),
                        lp[l]["lin_w"], lp[l]["lin_b"].reshape(1, H), wn)

    e_col, q_col, et, qt = _head(
        h, b32,
        params["lin1_w"], params["lin1_b"].reshape(1, H // 2),
        params["e_w"], params["e_b"].reshape(1, 1),
        params["q_w"], params["q_b"].reshape(1, 1),
    )
    return (e_col.reshape(N), q_col.reshape(N), et.reshape(NG), qt.reshape(NG))


# speedup vs baseline: 1.0273x; 1.0273x over previous
import functools
from math import pi as PI

import numpy as np
import jax
import jax.numpy as jnp
from jax import lax
from jax.experimental import pallas as pl
from jax.experimental.pallas import tpu as pltpu
from jax.experimental.pallas import tpu_sc as plsc

N = 10000
E = 320000
H = 128
F = 128
G = 50
L = 6
CUTOFF = 10.0
NG = 64

_NC = 2
_NS = 16
_NW = _NC * _NS
_EPW = E // _NW
_CH = 80
_NCHUNK = _EPW // _CH
_NP = 10240
_LOG2 = 0.6931471805599453
_STEP = CUTOFF / (G - 1)
_COEFF = -0.5 / _STEP ** 2

_mesh = plsc.VectorSubcoreMesh(
    core_axis_name="c", subcore_axis_name="s", num_cores=_NC, num_subcores=_NS
)


@functools.partial(
    pl.kernel,
    out_type=jax.ShapeDtypeStruct((E,), jnp.float32),
    mesh=_mesh,
    compiler_params=pltpu.CompilerParams(needs_layout_passes=False),
    scratch_types=[
        pltpu.VMEM((N,), jnp.float32),
        pltpu.VMEM((N,), jnp.float32),
        pltpu.VMEM((N,), jnp.float32),
        pltpu.VMEM((_EPW,), jnp.int32),
        pltpu.VMEM((_EPW,), jnp.int32),
        pltpu.VMEM((_EPW,), jnp.float32),
    ],
)
def _d2_kernel(posx_hbm, posy_hbm, posz_hbm, row_hbm, col_hbm, out_hbm,
               px, py, pz, rv, cv, dv):
    cid = lax.axis_index("c")
    sid = lax.axis_index("s")
    wid = sid * _NC + cid
    base = wid * _EPW
    pltpu.sync_copy(posx_hbm, px)
    pltpu.sync_copy(posy_hbm, py)
    pltpu.sync_copy(posz_hbm, pz)
    pltpu.sync_copy(row_hbm.at[pl.ds(base, _EPW)], rv)
    pltpu.sync_copy(col_hbm.at[pl.ds(base, _EPW)], cv)

    @plsc.parallel_loop(0, _EPW // 16, 1, unroll=2)
    def _(k):
        o = k * 16
        ir = rv[pl.ds(o, 16)]
        ic = cv[pl.ds(o, 16)]
        dx = plsc.load_gather(px, [ir]) - plsc.load_gather(px, [ic])
        dy = plsc.load_gather(py, [ir]) - plsc.load_gather(py, [ic])
        dz = plsc.load_gather(pz, [ir]) - plsc.load_gather(pz, [ic])
        dv[pl.ds(o, 16)] = dx * dx + dy * dy + dz * dz

    pltpu.sync_copy(dv, out_hbm.at[pl.ds(base, _EPW)])


@functools.partial(
    pl.kernel,
    out_type=jax.ShapeDtypeStruct((_NC, _NP, H), jnp.float32),
    mesh=_mesh,
    compiler_params=pltpu.CompilerParams(needs_layout_passes=False),
    scratch_types=[
        pltpu.VMEM_SHARED((_NP, H), jnp.float32),
        pltpu.VMEM((_CH,), jnp.int32),
        pltpu.VMEM((_CH,), jnp.int32),
        pltpu.VMEM((_CH,), jnp.int32),
        pltpu.VMEM((_CH,), jnp.int32),
        pltpu.VMEM((_CH, H // 2), jnp.int32),
        pltpu.VMEM((_CH, H // 2), jnp.int32),
        pltpu.VMEM((_CH, H), jnp.float32),
        pltpu.VMEM((_CH, H), jnp.float32),
        pltpu.SemaphoreType.DMA,
        pltpu.SemaphoreType.DMA,
        pltpu.SemaphoreType.DMA,
        pltpu.SemaphoreType.DMA,
        pltpu.SemaphoreType.DMA,
        pltpu.SemaphoreType.DMA,
        pltpu.SemaphoreType.DMA,
        pltpu.SemaphoreType.DMA,
    ],
)
def _cfconv(x1_hbm, wf_hbm, row_hbm, col_hbm, zeros_hbm, out_hbm,
            acc, rva, rvb, cva, cvb, wva, wvb, mva, mvb,
            sra, srb, sca, scb, swa, swb, sga, sgb):
    cid = lax.axis_index("c")
    sid = lax.axis_index("s")
    wid = sid * _NC + cid
    base = wid * _EPW
    rps = _NP // _NS
    pltpu.sync_copy(zeros_hbm.at[pl.ds(sid * rps, rps)],
                    acc.at[pl.ds(sid * rps, rps)])
    plsc.subcore_barrier()

    bufa = (rva, cva, wva, mva, sra, sca, swa, sga)
    bufb = (rvb, cvb, wvb, mvb, srb, scb, swb, sgb)

    def issue_data(t, buf):
        rv, cv, wv, mv, sr, sc, sw, sg = buf
        cb = base + t * _CH
        pltpu.async_copy(x1_hbm.at[rv], mv, sg)
        pltpu.async_copy(col_hbm.at[pl.ds(cb, _CH)], cv, sc)
        pltpu.async_copy(wf_hbm.at[pl.ds(cb, _CH)], wv, sw)

    def issue_row(t, buf):
        rv, cv, wv, mv, sr, sc, sw, sg = buf
        pltpu.async_copy(row_hbm.at[pl.ds(base + t * _CH, _CH)], rv, sr)

    def wait_row(buf):
        rv, cv, wv, mv, sr, sc, sw, sg = buf
        pltpu.make_async_copy(row_hbm.at[pl.ds(base, _CH)], rv, sr).wait()

    def phase(t, cur, nxt):
        rv, cv, wv, mv, sr, sc, sw, sg = cur

        @pl.when(t < _NCHUNK)
        def _():
            pltpu.make_async_copy(x1_hbm.at[rv], mv, sg).wait()
            pltpu.make_async_copy(col_hbm.at[pl.ds(base, _CH)], cv, sc).wait()
            pltpu.make_async_copy(wf_hbm.at[pl.ds(0, _CH)], wv, sw).wait()

            @pl.when(t + 1 < _NCHUNK)
            def _():
                wait_row(nxt)
                issue_data(t + 1, nxt)

            @pl.when(t + 2 < _NCHUNK)
            def _():
                issue_row(t + 2, cur)

            @plsc.parallel_loop(0, _CH, 1, unroll=2)
            def _(i):
                for j in range(H // 32):
                    wi = wv[i, pl.ds(j * 16, 16)]
                    wa = plsc.bitcast(wi << 16, jnp.float32)
                    wb = plsc.bitcast(wi & jnp.int32(-65536), jnp.float32)
                    s0 = pl.ds(j * 16, 16)
                    s1 = pl.ds(H // 2 + j * 16, 16)
                    mv[i, s0] = mv[i, s0] * wa
                    mv[i, s1] = mv[i, s1] * wb

            pltpu.sync_copy(mv, acc.at[cv], add=True)

    pltpu.sync_copy(row_hbm.at[pl.ds(base, _CH)], rva)
    issue_data(0, bufa)
    issue_row(1, bufb)

    def body(g, carry):
        phase(2 * g, bufa, bufb)
        phase(2 * g + 1, bufb, bufa)
        return carry

    lax.fori_loop(0, (_NCHUNK + 1) // 2, body, 0)
    plsc.subcore_barrier()
    pltpu.sync_copy(acc.at[pl.ds(sid * rps, rps)],
                    out_hbm.at[cid, pl.ds(sid * rps, rps)])


_BLKE = 640
_BLKN = 400
_ELR = E // 128
_BLKR = _ELR


def _prep_body(d2_ref, out_ref):
    ew = jnp.sqrt(d2_ref[:])
    out_ref[:] = 0.5 * (jnp.cos(ew * (PI / CUTOFF)) + 1.0)


def _prep(d2lanes):
    shp = (E // _BLKE, _BLKE // 128, 128)
    return pl.pallas_call(
        _prep_body,
        grid=(1,),
        in_specs=[pl.BlockSpec(shp, lambda i: (0, 0, 0))],
        out_specs=pl.BlockSpec(shp, lambda i: (0, 0, 0)),
        out_shape=jax.ShapeDtypeStruct(shp, jnp.float32),
    )(d2lanes)


def _tocol(blk):
    i0 = lax.broadcasted_iota(jnp.int32, (128, 128), 0)
    i1 = lax.broadcasted_iota(jnp.int32, (128, 128), 1)
    ident = (i0 == i1).astype(jnp.float32)
    parts = [lax.dot_general(ident, blk[s:s + 1, :], (((1,), (1,)), ((), ())))
             for s in range(_BLKE // 128)]
    return jnp.concatenate(parts, axis=0)


def _filters_body(d2_ref, c_ref, w1_ref, b1_ref, w2a_ref, b2a_ref,
                  w2b_ref, b2b_ref, out_ref):
    ew = _tocol(jnp.sqrt(d2_ref[0]))
    c = _tocol(c_ref[0])
    offs = lax.broadcasted_iota(jnp.int32, (1, G), 1).astype(jnp.float32) * _STEP
    ga = jnp.exp(_COEFF * (ew - offs) ** 2)
    t = jax.nn.softplus(ga @ w1_ref[:] + b1_ref[:]) - _LOG2
    wfa = ((t @ w2a_ref[:] + b2a_ref[:]) * c).astype(jnp.bfloat16)
    wfb = ((t @ w2b_ref[:] + b2b_ref[:]) * c).astype(jnp.bfloat16)
    lo = lax.bitcast_convert_type(wfa, jnp.uint16).astype(jnp.int32)
    hi = lax.bitcast_convert_type(wfb, jnp.uint16).astype(jnp.int32)
    out_ref[:] = lo | (hi << 16)


def _filters_one(d2lanes, cenvlanes, w1, b1, w2a, b2a, w2b, b2b):
    return pl.pallas_call(
        _filters_body,
        grid=(E // _BLKE,),
        in_specs=[
            pl.BlockSpec((1, _BLKE // 128, 128), lambda e: (e, 0, 0)),
            pl.BlockSpec((1, _BLKE // 128, 128), lambda e: (e, 0, 0)),
            pl.BlockSpec((G, F), lambda e: (0, 0)),
            pl.BlockSpec((1, F), lambda e: (0, 0)),
            pl.BlockSpec((F, F // 2), lambda e: (0, 0)),
            pl.BlockSpec((1, F // 2), lambda e: (0, 0)),
            pl.BlockSpec((F, F // 2), lambda e: (0, 0)),
            pl.BlockSpec((1, F // 2), lambda e: (0, 0)),
        ],
        out_specs=pl.BlockSpec((_BLKE, F // 2), lambda e: (e, 0)),
        out_shape=jax.ShapeDtypeStruct((E, F // 2), jnp.int32),
    )(d2lanes, cenvlanes, w1, b1, w2a, b2a, w2b, b2b)


def _embed_body(a_ref, emb_ref, w_ref, h_ref, x1_ref):
    a = a_ref[:]
    h0 = jnp.where(a == 0, emb_ref[0][None, :], emb_ref[1][None, :])
    h_ref[:] = h0
    x1_ref[:] = h0 @ w_ref[:]


def _embed(a32, emb, w0):
    return pl.pallas_call(
        _embed_body,
        grid=(N // _BLKN,),
        in_specs=[
            pl.BlockSpec((_BLKN, 1), lambda i: (i, 0)),
            pl.BlockSpec((2, H), lambda i: (0, 0)),
            pl.BlockSpec((H, F), lambda i: (0, 0)),
        ],
        out_specs=[
            pl.BlockSpec((_BLKN, H), lambda i: (i, 0)),
            pl.BlockSpec((_BLKN, F), lambda i: (i, 0)),
        ],
        out_shape=[
            jax.ShapeDtypeStruct((N, H), jnp.float32),
            jax.ShapeDtypeStruct((N, F), jnp.float32),
        ],
    )(a32, emb, w0)


def _update_body(h_ref, agg_ref, w2_ref, b2_ref, wl_ref, bl_ref, wn_ref,
                 h_out, x1_out):
    agg = agg_ref[0] + agg_ref[1]
    x2 = jax.nn.softplus(agg @ w2_ref[:] + b2_ref[:]) - _LOG2
    x2 = x2 @ wl_ref[:] + bl_ref[:]
    hn = h_ref[:] + x2
    h_out[:] = hn
    x1_out[:] = hn @ wn_ref[:]


def _update(h, agg2, w2, b2, wl, bl, wn):
    return pl.pallas_call(
        _update_body,
        grid=(N // _BLKN,),
        in_specs=[
            pl.BlockSpec((_BLKN, H), lambda i: (i, 0)),
            pl.BlockSpec((_NC, _BLKN, H), lambda i: (0, i, 0)),
            pl.BlockSpec((F, H), lambda i: (0, 0)),
            pl.BlockSpec((1, H), lambda i: (0, 0)),
            pl.BlockSpec((H, H), lambda i: (0, 0)),
            pl.BlockSpec((1, H), lambda i: (0, 0)),
            pl.BlockSpec((H, F), lambda i: (0, 0)),
        ],
        out_specs=[
            pl.BlockSpec((_BLKN, H), lambda i: (i, 0)),
            pl.BlockSpec((_BLKN, F), lambda i: (i, 0)),
        ],
        out_shape=[
            jax.ShapeDtypeStruct((N, H), jnp.float32),
            jax.ShapeDtypeStruct((N, F), jnp.float32),
        ],
    )(h, agg2, w2, b2, wl, bl, wn)


def _head_body(h_ref, b_ref, w1_ref, b1_ref, ew_ref, eb_ref, qw_ref, qb_ref,
               e_ref, q_ref, et_ref, qt_ref):
    i = pl.program_id(0)
    hh = jax.nn.softplus(h_ref[:] @ w1_ref[:] + b1_ref[:]) - _LOG2
    e = hh @ ew_ref[:] + eb_ref[:]
    q = hh @ qw_ref[:] + qb_ref[:]
    e_ref[:] = e
    q_ref[:] = q
    b = b_ref[:]
    onehot = (b == lax.broadcasted_iota(jnp.int32, (1, NG), 1)).astype(jnp.float32)
    et_p = lax.dot_general(onehot, e, (((0,), (0,)), ((), ())))
    qt_p = lax.dot_general(onehot, q, (((0,), (0,)), ((), ())))

    @pl.when(i == 0)
    def _():
        et_ref[:] = jnp.zeros_like(et_ref)
        qt_ref[:] = jnp.zeros_like(qt_ref)

    et_ref[:] += et_p
    qt_ref[:] += qt_p


def _head(h, b32, w1, b1, ew, eb, qw, qb):
    return pl.pallas_call(
        _head_body,
        grid=(N // _BLKN,),
        in_specs=[
            pl.BlockSpec((_BLKN, H), lambda i: (i, 0)),
            pl.BlockSpec((_BLKN, 1), lambda i: (i, 0)),
            pl.BlockSpec((H, H // 2), lambda i: (0, 0)),
            pl.BlockSpec((1, H // 2), lambda i: (0, 0)),
            pl.BlockSpec((H // 2, 1), lambda i: (0, 0)),
            pl.BlockSpec((1, 1), lambda i: (0, 0)),
            pl.BlockSpec((H // 2, 1), lambda i: (0, 0)),
            pl.BlockSpec((1, 1), lambda i: (0, 0)),
        ],
        out_specs=[
            pl.BlockSpec((_BLKN, 1), lambda i: (i, 0)),
            pl.BlockSpec((_BLKN, 1), lambda i: (i, 0)),
            pl.BlockSpec((NG, 1), lambda i: (0, 0)),
            pl.BlockSpec((NG, 1), lambda i: (0, 0)),
        ],
        out_shape=[
            jax.ShapeDtypeStruct((N, 1), jnp.float32),
            jax.ShapeDtypeStruct((N, 1), jnp.float32),
            jax.ShapeDtypeStruct((NG, 1), jnp.float32),
            jax.ShapeDtypeStruct((NG, 1), jnp.float32),
        ],
    )(h, b32, w1, b1, ew, eb, qw, qb)


def kernel(pos, params, atoms, batch_idx, edge_index):
    row = edge_index[0].astype(jnp.int32)
    col = edge_index[1].astype(jnp.int32)
    a32 = atoms.astype(jnp.int32).reshape(N, 1)
    b32 = batch_idx.astype(jnp.int32).reshape(N, 1)
    posx = pos[:, 0]
    posy = pos[:, 1]
    posz = pos[:, 2]

    d2 = _d2_kernel(posx, posy, posz, row, col)
    d2lanes = d2.reshape(E // _BLKE, _BLKE // 128, 128)
    cenvlanes = _prep(d2lanes)

    lp = params["layers"]
    h, x1 = _embed(a32, params["emb"], lp[0]["conv_lin1_w"])
    zeros_nh = jnp.zeros((_NP, H), jnp.float32)
    for l in range(L):
        w2 = lp[l]["mlp_w2"]
        b2 = lp[l]["mlp_b2"]
        wf = _filters_one(d2lanes, cenvlanes,
                          lp[l]["mlp_w1"], lp[l]["mlp_b1"].reshape(1, F),
                          w2[:, :F // 2], b2[:F // 2].reshape(1, F // 2),
                          w2[:, F // 2:], b2[F // 2:].reshape(1, F // 2))
        agg2 = _cfconv(x1, wf, row, col, zeros_nh)
        wn = (lp[l + 1]["conv_lin1_w"] if l < L - 1
              else jnp.zeros((H, F), jnp.float32))
        h, x1 = _update(h, agg2,
                        lp[l]["conv_lin2_w"], lp[l]["conv_lin2_b"].reshape(1, H),
                        lp[l]["lin_w"], lp[l]["lin_b"].reshape(1, H), wn)

    e_col, q_col, et, qt = _head(
        h, b32,
        params["lin1_w"], params["lin1_b"].reshape(1, H // 2),
        params["e_w"], params["e_b"].reshape(1, 1),
        params["q_w"], params["q_b"].reshape(1, 1),
    )
    return (e_col.reshape(N), q_col.reshape(N), et.reshape(NG), qt.reshape(NG))


# speedup vs baseline: 1.2606x; 1.2271x over previous
import functools
from math import pi as PI

import numpy as np
import jax
import jax.numpy as jnp
from jax import lax
from jax.experimental import pallas as pl
from jax.experimental.pallas import tpu as pltpu
from jax.experimental.pallas import tpu_sc as plsc

N = 10000
E = 320000
H = 128
F = 128
G = 50
L = 6
CUTOFF = 10.0
NG = 64

_NC = 2
_NS = 16
_NW = _NC * _NS
_EPW = E // _NW
_EH = E // 2
_PPW = _EH // _NW
_CH = 40
_NCHUNK = _PPW // _CH
_NP = 10240
_LOG2 = 0.6931471805599453
_STEP = CUTOFF / (G - 1)
_COEFF = -0.5 / _STEP ** 2

_mesh = plsc.VectorSubcoreMesh(
    core_axis_name="c", subcore_axis_name="s", num_cores=_NC, num_subcores=_NS
)


@functools.partial(
    pl.kernel,
    out_type=jax.ShapeDtypeStruct((E,), jnp.float32),
    mesh=_mesh,
    compiler_params=pltpu.CompilerParams(needs_layout_passes=False),
    scratch_types=[
        pltpu.VMEM((N,), jnp.float32),
        pltpu.VMEM((N,), jnp.float32),
        pltpu.VMEM((N,), jnp.float32),
        pltpu.VMEM((_EPW,), jnp.int32),
        pltpu.VMEM((_EPW,), jnp.int32),
        pltpu.VMEM((_EPW,), jnp.float32),
    ],
)
def _d2_kernel(posx_hbm, posy_hbm, posz_hbm, row_hbm, col_hbm, out_hbm,
               px, py, pz, rv, cv, dv):
    cid = lax.axis_index("c")
    sid = lax.axis_index("s")
    wid = sid * _NC + cid
    base = wid * _EPW
    pltpu.sync_copy(posx_hbm, px)
    pltpu.sync_copy(posy_hbm, py)
    pltpu.sync_copy(posz_hbm, pz)
    pltpu.sync_copy(row_hbm.at[pl.ds(base, _EPW)], rv)
    pltpu.sync_copy(col_hbm.at[pl.ds(base, _EPW)], cv)

    @plsc.parallel_loop(0, _EPW // 16, 1, unroll=2)
    def _(k):
        o = k * 16
        ir = rv[pl.ds(o, 16)]
        ic = cv[pl.ds(o, 16)]
        dx = plsc.load_gather(px, [ir]) - plsc.load_gather(px, [ic])
        dy = plsc.load_gather(py, [ir]) - plsc.load_gather(py, [ic])
        dz = plsc.load_gather(pz, [ir]) - plsc.load_gather(pz, [ic])
        dv[pl.ds(o, 16)] = dx * dx + dy * dy + dz * dz

    pltpu.sync_copy(dv, out_hbm.at[pl.ds(base, _EPW)])


@functools.partial(
    pl.kernel,
    out_type=jax.ShapeDtypeStruct((_NC, _NP, H), jnp.float32),
    mesh=_mesh,
    compiler_params=pltpu.CompilerParams(needs_layout_passes=False),
    scratch_types=[
        pltpu.VMEM_SHARED((_NP, H), jnp.float32),
        pltpu.VMEM((2, _CH), jnp.int32),
        pltpu.VMEM((2, _CH), jnp.int32),
        pltpu.VMEM((2, _CH), jnp.int32),
        pltpu.VMEM((2, _CH), jnp.int32),
        pltpu.VMEM((_CH, H), jnp.int32),
        pltpu.VMEM((_CH, H), jnp.int32),
        pltpu.VMEM((_CH, H), jnp.float32),
        pltpu.VMEM((_CH, H), jnp.float32),
        pltpu.VMEM((_CH, H), jnp.float32),
        pltpu.VMEM((_CH, H), jnp.float32),
        pltpu.SemaphoreType.DMA,
        pltpu.SemaphoreType.DMA,
        pltpu.SemaphoreType.DMA,
        pltpu.SemaphoreType.DMA,
        pltpu.SemaphoreType.DMA,
        pltpu.SemaphoreType.DMA,
        pltpu.SemaphoreType.DMA,
        pltpu.SemaphoreType.DMA,
    ],
)
def _cfconv(x1_hbm, wf_hbm, row_hbm, col_hbm, zeros_hbm, out_hbm,
            acc, rva, rvb, cva, cvb, wva, wvb, mla, mlb, mha, mhb,
            sia, sib, swa, swb, sla, slb, sha, shb):
    cid = lax.axis_index("c")
    sid = lax.axis_index("s")
    wid = sid * _NC + cid
    base = wid * _PPW
    rps = _NP // _NS
    pltpu.sync_copy(zeros_hbm.at[pl.ds(sid * rps, rps)],
                    acc.at[pl.ds(sid * rps, rps)])
    plsc.subcore_barrier()

    bufa = (rva, cva, wva, mla, mha, sia, swa, sla, sha)
    bufb = (rvb, cvb, wvb, mlb, mhb, sib, swb, slb, shb)

    def issue_idx(t, buf):
        rv, cv, wv, ml, mh, si, sw, sl, sh = buf
        cb = base + t * _CH
        pltpu.async_copy(row_hbm.at[pl.ds(cb, _CH)], rv.at[0], si)
        pltpu.async_copy(row_hbm.at[pl.ds(cb + _EH, _CH)], rv.at[1], si)
        pltpu.async_copy(col_hbm.at[pl.ds(cb, _CH)], cv.at[0], si)
        pltpu.async_copy(col_hbm.at[pl.ds(cb + _EH, _CH)], cv.at[1], si)

    def wait_idx(buf):
        rv, cv, wv, ml, mh, si, sw, sl, sh = buf
        for ref in (rv.at[0], rv.at[1], cv.at[0], cv.at[1]):
            pltpu.make_async_copy(row_hbm.at[pl.ds(base, _CH)], ref, si).wait()

    def issue_data(t, buf):
        rv, cv, wv, ml, mh, si, sw, sl, sh = buf
        pltpu.async_copy(x1_hbm.at[rv.at[0]], ml, sl)
        pltpu.async_copy(x1_hbm.at[rv.at[1]], mh, sh)
        pltpu.async_copy(wf_hbm.at[pl.ds(base + t * _CH, _CH)], wv, sw)

    def phase(t, cur, nxt):
        rv, cv, wv, ml, mh, si, sw, sl, sh = cur

        @pl.when(t < _NCHUNK)
        def _():
            pltpu.make_async_copy(x1_hbm.at[rv.at[0]], ml, sl).wait()
            pltpu.make_async_copy(x1_hbm.at[rv.at[1]], mh, sh).wait()
            pltpu.make_async_copy(wf_hbm.at[pl.ds(base, _CH)], wv, sw).wait()

            @pl.when(t + 1 < _NCHUNK)
            def _():
                wait_idx(nxt)
                issue_data(t + 1, nxt)

            @plsc.parallel_loop(0, _CH, 1, unroll=2)
            def _(i):
                for j in range(H // 16):
                    s = pl.ds(j * 16, 16)
                    wi = wv[i, s]
                    wa = plsc.bitcast(wi << 16, jnp.float32)
                    wb = plsc.bitcast(wi & jnp.int32(-65536), jnp.float32)
                    ml[i, s] = ml[i, s] * wa
                    mh[i, s] = mh[i, s] * wb

            pltpu.sync_copy(ml, acc.at[cv.at[0]], add=True)
            pltpu.sync_copy(mh, acc.at[cv.at[1]], add=True)

            @pl.when(t + 2 < _NCHUNK)
            def _():
                issue_idx(t + 2, cur)

    issue_idx(0, bufa)
    wait_idx(bufa)
    issue_data(0, bufa)
    issue_idx(1, bufb)

    def body(g, carry):
        phase(2 * g, bufa, bufb)
        phase(2 * g + 1, bufb, bufa)
        return carry

    lax.fori_loop(0, (_NCHUNK + 1) // 2, body, 0)
    plsc.subcore_barrier()
    pltpu.sync_copy(acc.at[pl.ds(sid * rps, rps)],
                    out_hbm.at[cid, pl.ds(sid * rps, rps)])


_BLKE = 640
_BLKN = 400
_ELR = E // 128
_BLKR = _ELR


def _prep_body(d2_ref, out_ref):
    ew = jnp.sqrt(d2_ref[:])
    out_ref[:] = 0.5 * (jnp.cos(ew * (PI / CUTOFF)) + 1.0)


def _prep(d2lanes):
    shp = (E // _BLKE, _BLKE // 128, 128)
    return pl.pallas_call(
        _prep_body,
        grid=(1,),
        in_specs=[pl.BlockSpec(shp, lambda i: (0, 0, 0))],
        out_specs=pl.BlockSpec(shp, lambda i: (0, 0, 0)),
        out_shape=jax.ShapeDtypeStruct(shp, jnp.float32),
    )(d2lanes)


def _tocol(blk):
    i0 = lax.broadcasted_iota(jnp.int32, (128, 128), 0)
    i1 = lax.broadcasted_iota(jnp.int32, (128, 128), 1)
    ident = (i0 == i1).astype(jnp.float32)
    parts = [lax.dot_general(ident, blk[s:s + 1, :], (((1,), (1,)), ((), ())))
             for s in range(_BLKE // 128)]
    return jnp.concatenate(parts, axis=0)


def _filters_half(d2blk, cblk, w1, b1, w2, b2):
    ew = _tocol(jnp.sqrt(d2blk))
    c = _tocol(cblk)
    offs = lax.broadcasted_iota(jnp.int32, (1, G), 1).astype(jnp.float32) * _STEP
    ga = jnp.exp(_COEFF * (ew - offs) ** 2)
    t = jax.nn.softplus(ga @ w1 + b1) - _LOG2
    return ((t @ w2 + b2) * c).astype(jnp.bfloat16)


def _filters_body(d2a_ref, ca_ref, d2b_ref, cb_ref, w1_ref, b1_ref,
                  w2_ref, b2_ref, out_ref):
    w1 = w1_ref[:]
    b1 = b1_ref[:]
    w2 = w2_ref[:]
    b2 = b2_ref[:]
    wfa = _filters_half(d2a_ref[0], ca_ref[0], w1, b1, w2, b2)
    wfb = _filters_half(d2b_ref[0], cb_ref[0], w1, b1, w2, b2)
    lo = lax.bitcast_convert_type(wfa, jnp.uint16).astype(jnp.int32)
    hi = lax.bitcast_convert_type(wfb, jnp.uint16).astype(jnp.int32)
    out_ref[:] = lo | (hi << 16)


_NBH = _EH // _BLKE


def _filters_one(d2lanes, cenvlanes, w1, b1, w2, b2):
    return pl.pallas_call(
        _filters_body,
        grid=(_NBH,),
        in_specs=[
            pl.BlockSpec((1, _BLKE // 128, 128), lambda e: (e, 0, 0)),
            pl.BlockSpec((1, _BLKE // 128, 128), lambda e: (e, 0, 0)),
            pl.BlockSpec((1, _BLKE // 128, 128), lambda e: (e + _NBH, 0, 0)),
            pl.BlockSpec((1, _BLKE // 128, 128), lambda e: (e + _NBH, 0, 0)),
            pl.BlockSpec((G, F), lambda e: (0, 0)),
            pl.BlockSpec((1, F), lambda e: (0, 0)),
            pl.BlockSpec((F, F), lambda e: (0, 0)),
            pl.BlockSpec((1, F), lambda e: (0, 0)),
        ],
        out_specs=pl.BlockSpec((_BLKE, F), lambda e: (e, 0)),
        out_shape=jax.ShapeDtypeStruct((_EH, F), jnp.int32),
    )(d2lanes, cenvlanes, d2lanes, cenvlanes, w1, b1, w2, b2)


def _embed_body(a_ref, emb_ref, w_ref, h_ref, x1_ref):
    a = a_ref[:]
    h0 = jnp.where(a == 0, emb_ref[0][None, :], emb_ref[1][None, :])
    h_ref[:] = h0
    x1_ref[:] = h0 @ w_ref[:]


def _embed(a32, emb, w0):
    return pl.pallas_call(
        _embed_body,
        grid=(N // _BLKN,),
        in_specs=[
            pl.BlockSpec((_BLKN, 1), lambda i: (i, 0)),
            pl.BlockSpec((2, H), lambda i: (0, 0)),
            pl.BlockSpec((H, F), lambda i: (0, 0)),
        ],
        out_specs=[
            pl.BlockSpec((_BLKN, H), lambda i: (i, 0)),
            pl.BlockSpec((_BLKN, F), lambda i: (i, 0)),
        ],
        out_shape=[
            jax.ShapeDtypeStruct((N, H), jnp.float32),
            jax.ShapeDtypeStruct((N, F), jnp.float32),
        ],
    )(a32, emb, w0)


def _update_body(h_ref, agg_ref, w2_ref, b2_ref, wl_ref, bl_ref, wn_ref,
                 h_out, x1_out):
    agg = agg_ref[0] + agg_ref[1]
    x2 = jax.nn.softplus(agg @ w2_ref[:] + b2_ref[:]) - _LOG2
    x2 = x2 @ wl_ref[:] + bl_ref[:]
    hn = h_ref[:] + x2
    h_out[:] = hn
    x1_out[:] = hn @ wn_ref[:]


def _update(h, agg2, w2, b2, wl, bl, wn):
    return pl.pallas_call(
        _update_body,
        grid=(N // _BLKN,),
        in_specs=[
            pl.BlockSpec((_BLKN, H), lambda i: (i, 0)),
            pl.BlockSpec((_NC, _BLKN, H), lambda i: (0, i, 0)),
            pl.BlockSpec((F, H), lambda i: (0, 0)),
            pl.BlockSpec((1, H), lambda i: (0, 0)),
            pl.BlockSpec((H, H), lambda i: (0, 0)),
            pl.BlockSpec((1, H), lambda i: (0, 0)),
            pl.BlockSpec((H, F), lambda i: (0, 0)),
        ],
        out_specs=[
            pl.BlockSpec((_BLKN, H), lambda i: (i, 0)),
            pl.BlockSpec((_BLKN, F), lambda i: (i, 0)),
        ],
        out_shape=[
            jax.ShapeDtypeStruct((N, H), jnp.float32),
            jax.ShapeDtypeStruct((N, F), jnp.float32),
        ],
    )(h, agg2, w2, b2, wl, bl, wn)


def _head_body(h_ref, b_ref, w1_ref, b1_ref, ew_ref, eb_ref, qw_ref, qb_ref,
               e_ref, q_ref, et_ref, qt_ref):
    i = pl.program_id(0)
    hh = jax.nn.softplus(h_ref[:] @ w1_ref[:] + b1_ref[:]) - _LOG2
    e = hh @ ew_ref[:] + eb_ref[:]
    q = hh @ qw_ref[:] + qb_ref[:]
    e_ref[:] = e
    q_ref[:] = q
    b = b_ref[:]
    onehot = (b == lax.broadcasted_iota(jnp.int32, (1, NG), 1)).astype(jnp.float32)
    et_p = lax.dot_general(onehot, e, (((0,), (0,)), ((), ())))
    qt_p = lax.dot_general(onehot, q, (((0,), (0,)), ((), ())))

    @pl.when(i == 0)
    def _():
        et_ref[:] = jnp.zeros_like(et_ref)
        qt_ref[:] = jnp.zeros_like(qt_ref)

    et_ref[:] += et_p
    qt_ref[:] += qt_p


def _head(h, b32, w1, b1, ew, eb, qw, qb):
    return pl.pallas_call(
        _head_body,
        grid=(N // _BLKN,),
        in_specs=[
            pl.BlockSpec((_BLKN, H), lambda i: (i, 0)),
            pl.BlockSpec((_BLKN, 1), lambda i: (i, 0)),
            pl.BlockSpec((H, H // 2), lambda i: (0, 0)),
            pl.BlockSpec((1, H // 2), lambda i: (0, 0)),
            pl.BlockSpec((H // 2, 1), lambda i: (0, 0)),
            pl.BlockSpec((1, 1), lambda i: (0, 0)),
            pl.BlockSpec((H // 2, 1), lambda i: (0, 0)),
            pl.BlockSpec((1, 1), lambda i: (0, 0)),
        ],
        out_specs=[
            pl.BlockSpec((_BLKN, 1), lambda i: (i, 0)),
            pl.BlockSpec((_BLKN, 1), lambda i: (i, 0)),
            pl.BlockSpec((NG, 1), lambda i: (0, 0)),
            pl.BlockSpec((NG, 1), lambda i: (0, 0)),
        ],
        out_shape=[
            jax.ShapeDtypeStruct((N, 1), jnp.float32),
            jax.ShapeDtypeStruct((N, 1), jnp.float32),
            jax.ShapeDtypeStruct((NG, 1), jnp.float32),
            jax.ShapeDtypeStruct((NG, 1), jnp.float32),
        ],
    )(h, b32, w1, b1, ew, eb, qw, qb)


def kernel(pos, params, atoms, batch_idx, edge_index):
    row = edge_index[0].astype(jnp.int32)
    col = edge_index[1].astype(jnp.int32)
    a32 = atoms.astype(jnp.int32).reshape(N, 1)
    b32 = batch_idx.astype(jnp.int32).reshape(N, 1)
    posx = pos[:, 0]
    posy = pos[:, 1]
    posz = pos[:, 2]

    d2 = _d2_kernel(posx, posy, posz, row, col)
    d2lanes = d2.reshape(E // _BLKE, _BLKE // 128, 128)
    cenvlanes = _prep(d2lanes)

    lp = params["layers"]
    h, x1 = _embed(a32, params["emb"], lp[0]["conv_lin1_w"])
    zeros_nh = jnp.zeros((_NP, H), jnp.float32)
    for l in range(L):
        wf = _filters_one(d2lanes, cenvlanes,
                          lp[l]["mlp_w1"], lp[l]["mlp_b1"].reshape(1, F),
                          lp[l]["mlp_w2"], lp[l]["mlp_b2"].reshape(1, F))
        agg2 = _cfconv(x1, wf, row, col, zeros_nh)
        wn = (lp[l + 1]["conv_lin1_w"] if l < L - 1
              else jnp.zeros((H, F), jnp.float32))
        h, x1 = _update(h, agg2,
                        lp[l]["conv_lin2_w"], lp[l]["conv_lin2_b"].reshape(1, H),
                        lp[l]["lin_w"], lp[l]["lin_b"].reshape(1, H), wn)

    e_col, q_col, et, qt = _head(
        h, b32,
        params["lin1_w"], params["lin1_b"].reshape(1, H // 2),
        params["e_w"], params["e_b"].reshape(1, 1),
        params["q_w"], params["q_b"].reshape(1, 1),
    )
    return (e_col.reshape(N), q_col.reshape(N), et.reshape(NG), qt.reshape(NG))
